# Initial kernel scaffold; baseline (speedup 1.0000x reference)
import jax
import jax.numpy as jnp
from jax.experimental import pallas as pl


def kernel(x, edge_index, W, b):
    raise NotImplementedError("write your pallas kernel here")


# speedup vs baseline: 19.0195x; 19.0195x over previous
import functools

import jax
import jax.numpy as jnp
from jax import lax
from jax.experimental import pallas as pl
from jax.experimental.pallas import tpu as pltpu
from jax.experimental.pallas import tpu_sc as plsc

_LANES = 16


def _mm_kernel(x_ref, w_ref, u_ref, v_ref):
    xb = x_ref[...]
    w = w_ref[...]
    c = w.shape[0]
    w1 = w[:, :c]
    w2 = w[:, c:]
    dn = (((1,), (1,)), ((), ()))
    u_ref[...] = lax.dot_general(xb, w1 - w2, dn, preferred_element_type=jnp.float32)
    v_ref[...] = lax.dot_general(xb, w2, dn, preferred_element_type=jnp.float32)


def _node_tables(xt, w, nt):
    g, c = xt.shape
    grid = g // nt
    return pl.pallas_call(
        _mm_kernel,
        grid=(grid,),
        in_specs=[
            pl.BlockSpec((nt, c), lambda i: (i, 0)),
            pl.BlockSpec((c, 2 * c), lambda i: (0, 0)),
        ],
        out_specs=[pl.BlockSpec((nt, c), lambda i: (i, 0))] * 2,
        out_shape=[jax.ShapeDtypeStruct((g, c), jnp.float32)] * 2,
    )(xt, w)


def _make_edge_max(g_pad, c, k, nb):
    info = plsc.get_sparse_core_info()
    nc, ns = info.num_cores, info.num_subcores
    nw = nc * ns
    npw = g_pad // nw
    nblk = npw // nb
    mesh = plsc.VectorSubcoreMesh(core_axis_name="c", subcore_axis_name="s")

    @functools.partial(
        pl.kernel,
        mesh=mesh,
        out_type=jax.ShapeDtypeStruct((g_pad, c), jnp.float32),
        scratch_types=[
            pltpu.VMEM((nb * k,), jnp.int32),
            pltpu.VMEM((nb * k,), jnp.int32),
            pltpu.VMEM((nb * k, c), jnp.float32),
            pltpu.VMEM((nb * k, c), jnp.float32),
            pltpu.VMEM((c,), jnp.float32),
            pltpu.VMEM((nb, c), jnp.float32),
            pltpu.SemaphoreType.DMA,
            pltpu.SemaphoreType.DMA,
        ],
    )
    def edge_max(u_hbm, v_hbm, ii_hbm, jj_hbm, b_hbm, out_hbm,
                 ii_v, jj_v, ur_v, vr_v, b_v, ob_v, sem_u, sem_v):
        wid = lax.axis_index("s") * nc + lax.axis_index("c")
        base = wid * npw
        pltpu.sync_copy(b_hbm, b_v)

        def blk(i, carry):
            nb0 = base + i * nb
            pltpu.sync_copy(ii_hbm.at[pl.ds(nb0 * k, nb * k)], ii_v)
            pltpu.sync_copy(jj_hbm.at[pl.ds(nb0 * k, nb * k)], jj_v)
            cu = pltpu.async_copy(u_hbm.at[ii_v], ur_v, sem_u)
            cv = pltpu.async_copy(v_hbm.at[jj_v], vr_v, sem_v)
            cu.wait()
            cv.wait()

            def node(n, ncarry):
                for c16 in range(c // _LANES):
                    sl = pl.ds(c16 * _LANES, _LANES)
                    acc = ur_v[n * k, sl] + vr_v[n * k, sl]
                    for kk in range(1, k):
                        acc = jnp.maximum(acc, ur_v[n * k + kk, sl] + vr_v[n * k + kk, sl])
                    ob_v[n, sl] = jnp.maximum(acc + b_v[sl], 0.0)
                return ncarry

            lax.fori_loop(0, nb, node, 0)
            pltpu.sync_copy(ob_v, out_hbm.at[pl.ds(nb0, nb)])
            return carry

        lax.fori_loop(0, nblk, blk, 0)

    return edge_max


def kernel(x, edge_index, W, b):
    bsz, c, n, _ = x.shape
    kk = edge_index.shape[-1]
    g = bsz * n

    xt = jnp.transpose(x[:, :, :, 0], (0, 2, 1)).reshape(g, c)

    u, v = _node_tables(xt, W, nt=2000)

    offs = (jnp.arange(bsz, dtype=jnp.int32) * n)[:, None, None]
    idx_i = (edge_index[1] + offs).reshape(-1)
    idx_j = (edge_index[0] + offs).reshape(-1)

    nb = 8
    nw = 32
    g_pad = ((g + nw * nb - 1) // (nw * nb)) * (nw * nb)
    pad = g_pad - g
    if pad:
        zp = jnp.zeros((pad * kk,), jnp.int32)
        idx_i = jnp.concatenate([idx_i, zp])
        idx_j = jnp.concatenate([idx_j, zp])

    edge_max = _make_edge_max(g_pad, c, kk, nb)
    o_pad = edge_max(u, v, idx_i, idx_j, b)

    out = o_pad[:g].reshape(bsz, n, c).transpose(0, 2, 1)[..., None]
    return out
